# baseline (speedup 1.0000x reference)
import functools

import jax
import jax.numpy as jnp
from jax.experimental import pallas as pl
from jax.experimental.pallas import tpu as pltpu

B = 16384
J_IN = 45
J_OUT = 25
CH = 3
LC = 2048
NK = B // LC

MODE = "matmul"


def _body(jm_ref, x_ref, o_ref):
    x = x_ref[0]
    jm = jm_ref[0]
    if MODE == "take":
        idx = jnp.broadcast_to(jm[:, None], (J_OUT, LC))
        o_ref[0] = jnp.take_along_axis(x, idx, axis=0)
    else:
        sel = (jm[:, None] == jax.lax.broadcasted_iota(jnp.int32, (J_OUT, J_IN), 1)
               ).astype(jnp.float32)
        o_ref[0] = jnp.dot(sel, x, preferred_element_type=jnp.float32)


@jax.jit
def _tc_call(jm, xt):
    return pl.pallas_call(
        _body,
        grid=(CH, NK),
        in_specs=[
            pl.BlockSpec((1, J_OUT), lambda c, k: (0, 0)),
            pl.BlockSpec((1, J_IN, LC), lambda c, k: (c, 0, k)),
        ],
        out_specs=pl.BlockSpec((1, J_OUT, LC), lambda c, k: (c, 0, k)),
        out_shape=jax.ShapeDtypeStruct((CH, J_OUT, B), jnp.float32),
    )(jm, xt)


def kernel(joints, joint_maps):
    xt = jnp.transpose(joints, (2, 1, 0))
    jm = joint_maps.astype(jnp.int32).reshape(1, J_OUT)
    out_t = _tc_call(jm, xt)
    return jnp.transpose(out_t, (2, 1, 0))


# speedup vs baseline: 2.1242x; 2.1242x over previous
import functools

import jax
import jax.numpy as jnp
from jax.experimental import pallas as pl
from jax.experimental.pallas import tpu as pltpu

B = 16384
J_IN = 45
J_OUT = 25
CH = 3
LC = 8192
NK = B // LC

MODE = "matmul"


def _body(jm_ref, x_ref, o_ref):
    x = x_ref[0]
    jm = jm_ref[0]
    if MODE == "take":
        idx = jnp.broadcast_to(jm[:, None], (J_OUT, LC))
        o_ref[0] = jnp.take_along_axis(x, idx, axis=0)
    else:
        sel = (jm[:, None] == jax.lax.broadcasted_iota(jnp.int32, (J_OUT, J_IN), 1)
               ).astype(jnp.float32)
        o_ref[0] = jnp.dot(sel, x, preferred_element_type=jnp.float32)


@jax.jit
def _tc_call(jm, xt):
    return pl.pallas_call(
        _body,
        grid=(CH, NK),
        in_specs=[
            pl.BlockSpec((1, J_OUT), lambda c, k: (0, 0)),
            pl.BlockSpec((1, J_IN, LC), lambda c, k: (c, 0, k)),
        ],
        out_specs=pl.BlockSpec((1, J_OUT, LC), lambda c, k: (c, 0, k)),
        out_shape=jax.ShapeDtypeStruct((CH, J_OUT, B), jnp.float32),
    )(jm, xt)


def kernel(joints, joint_maps):
    xt = jnp.transpose(joints, (2, 1, 0))
    jm = joint_maps.astype(jnp.int32).reshape(1, J_OUT)
    out_t = _tc_call(jm, xt)
    return jnp.transpose(out_t, (2, 1, 0))


# speedup vs baseline: 2.3708x; 1.1161x over previous
import functools

import jax
import jax.numpy as jnp
from jax.experimental import pallas as pl
from jax.experimental.pallas import tpu as pltpu

B = 16384
J_IN = 45
J_OUT = 25
CH = 3
LC = 16384
NK = B // LC

MODE = "matmul"


def _body(jm_ref, x_ref, o_ref):
    x = x_ref[0]
    jm = jm_ref[0]
    if MODE == "take":
        idx = jnp.broadcast_to(jm[:, None], (J_OUT, LC))
        o_ref[0] = jnp.take_along_axis(x, idx, axis=0)
    else:
        sel = (jm[:, None] == jax.lax.broadcasted_iota(jnp.int32, (J_OUT, J_IN), 1)
               ).astype(jnp.float32)
        o_ref[0] = jnp.dot(sel, x, preferred_element_type=jnp.float32)


@jax.jit
def _tc_call(jm, xt):
    return pl.pallas_call(
        _body,
        grid=(CH, NK),
        in_specs=[
            pl.BlockSpec((1, J_OUT), lambda c, k: (0, 0)),
            pl.BlockSpec((1, J_IN, LC), lambda c, k: (c, 0, k)),
        ],
        out_specs=pl.BlockSpec((1, J_OUT, LC), lambda c, k: (c, 0, k)),
        out_shape=jax.ShapeDtypeStruct((CH, J_OUT, B), jnp.float32),
    )(jm, xt)


def kernel(joints, joint_maps):
    xt = jnp.transpose(joints, (2, 1, 0))
    jm = joint_maps.astype(jnp.int32).reshape(1, J_OUT)
    out_t = _tc_call(jm, xt)
    return jnp.transpose(out_t, (2, 1, 0))


# speedup vs baseline: 2.4918x; 1.0510x over previous
import functools

import jax
import jax.numpy as jnp
from jax.experimental import pallas as pl
from jax.experimental.pallas import tpu as pltpu

B = 16384
J_IN = 45
J_OUT = 25
CH = 3
LC = 16384
NK = B // LC

MODE = "matmul"


def _body(jm_ref, x_ref, o_ref):
    jm = jm_ref[0]
    sel = (jm[:, None] == jax.lax.broadcasted_iota(jnp.int32, (J_OUT, J_IN), 1)
           ).astype(jnp.float32)
    for c in range(CH):
        o_ref[c] = jnp.dot(sel, x_ref[c], preferred_element_type=jnp.float32)


@jax.jit
def _tc_call(jm, xt):
    return pl.pallas_call(
        _body,
        grid=(1,),
        in_specs=[
            pl.BlockSpec((1, J_OUT), lambda i: (0, 0)),
            pl.BlockSpec((CH, J_IN, LC), lambda i: (0, 0, 0)),
        ],
        out_specs=pl.BlockSpec((CH, J_OUT, LC), lambda i: (0, 0, 0)),
        out_shape=jax.ShapeDtypeStruct((CH, J_OUT, B), jnp.float32),
    )(jm, xt)


def kernel(joints, joint_maps):
    xt = jnp.transpose(joints, (2, 1, 0))
    jm = joint_maps.astype(jnp.int32).reshape(1, J_OUT)
    out_t = _tc_call(jm, xt)
    return jnp.transpose(out_t, (2, 1, 0))


# speedup vs baseline: 2.9744x; 1.1937x over previous
import functools

import jax
import jax.numpy as jnp
from jax.experimental import pallas as pl
from jax.experimental.pallas import tpu as pltpu

B = 16384
J_IN = 45
J_OUT = 25
CH = 3
LC = 16384
NK = B // LC

MODE = "matmul"


def _body(jm_ref, x_ref, o_ref):
    jm = jm_ref[0]
    sel = (jm[:, None] == jax.lax.broadcasted_iota(jnp.int32, (J_OUT, J_IN), 1)
           ).astype(jnp.float32)
    for c in range(CH):
        o_ref[c] = jnp.dot(sel, x_ref[c], preferred_element_type=jnp.float32)


@jax.jit
def _tc_call(jm, xt):
    return pl.pallas_call(
        _body,
        grid=(2,),
        in_specs=[
            pl.BlockSpec((1, J_OUT), lambda k: (0, 0)),
            pl.BlockSpec((CH, J_IN, LC // 2), lambda k: (0, 0, k)),
        ],
        out_specs=pl.BlockSpec((CH, J_OUT, LC // 2), lambda k: (0, 0, k)),
        out_shape=jax.ShapeDtypeStruct((CH, J_OUT, B), jnp.float32),
    )(jm, xt)


def kernel(joints, joint_maps):
    xt = jnp.transpose(joints, (2, 1, 0))
    jm = joint_maps.astype(jnp.int32).reshape(1, J_OUT)
    out_t = _tc_call(jm, xt)
    return jnp.transpose(out_t, (2, 1, 0))
